# Initial kernel scaffold; baseline (speedup 1.0000x reference)
import jax
import jax.numpy as jnp
from jax.experimental import pallas as pl


def kernel(boxes, scores):
    raise NotImplementedError("write your pallas kernel here")


# speedup vs baseline: 28.3766x; 28.3766x over previous
import jax
import jax.numpy as jnp
from jax.experimental import pallas as pl
from jax.experimental.pallas import tpu as pltpu

_SCORE_THRESH = 0.05
_NUM_CAND = 1000
_PAD = 1024
_NMS_THRESH = 0.5
_MAX_DET = 100
_NEG = -1e9


def _nms_kernel(bc_ref, bt_ref, sr_ref, out_ref, iou_ref):
    f32 = jnp.float32
    x1c = bc_ref[:, 0:1]
    y1c = bc_ref[:, 1:2]
    x2c = bc_ref[:, 2:3]
    y2c = bc_ref[:, 3:4]
    x1r = bt_ref[0:1, :]
    y1r = bt_ref[1:2, :]
    x2r = bt_ref[2:3, :]
    y2r = bt_ref[3:4, :]

    area_c = (x2c - x1c) * (y2c - y1c)
    area_r = (x2r - x1r) * (y2r - y1r)
    w = jnp.maximum(jnp.minimum(x2c, x2r) - jnp.maximum(x1c, x1r), 0.0)
    h = jnp.maximum(jnp.minimum(y2c, y2r) - jnp.maximum(y1c, y1r), 0.0)
    inter = w * h
    union = area_c + area_r - inter
    iou_ref[...] = inter / jnp.maximum(union, 1e-9)

    scores = sr_ref[...]
    idx = jax.lax.broadcasted_iota(jnp.int32, (1, _PAD), 1)
    valid = (scores > -1e8).astype(f32)

    def body(i, keep):
        row = iou_ref[pl.ds(i, 1), :]
        keep_i = jnp.max(jnp.where(idx == i, keep, 0.0))
        sup = (row > _NMS_THRESH) & (idx > i)
        return jnp.where(sup & (keep_i > 0.0), 0.0, keep)

    keep = jax.lax.fori_loop(0, _NUM_CAND, body, valid)

    r_i = jax.lax.broadcasted_iota(jnp.int32, (_PAD, _PAD), 0)
    c_i = jax.lax.broadcasted_iota(jnp.int32, (_PAD, _PAD), 1)
    tri = (r_i <= c_i).astype(f32)
    dn = (((1,), (0,)), ((), ()))
    ck = jax.lax.dot_general(keep, tri, dn, preferred_element_type=f32)
    cn = jax.lax.dot_general(1.0 - keep, tri, dn, preferred_element_type=f32)
    num_keep = jnp.max(ck)
    pos = jnp.where(keep > 0.0, ck - 1.0, num_keep + cn - 1.0)
    posi = pos.astype(jnp.int32)

    o_i = jax.lax.broadcasted_iota(jnp.int32, (128, _PAD), 0)
    onehot = (posi == o_i).astype(f32)
    dnt = (((1,), (1,)), ((), ()))
    out_boxes = jax.lax.dot_general(onehot, bt_ref[...], dnt,
                                    preferred_element_type=f32)
    fs = jnp.where(keep > 0.0, scores, _NEG)
    out_scores = jax.lax.dot_general(onehot, fs, dnt,
                                     preferred_element_type=f32)
    col = jax.lax.broadcasted_iota(jnp.int32, (128, 8), 1)
    out_ref[...] = jnp.where(col < 4, out_boxes,
                             jnp.where(col == 4, out_scores, 0.0))


def kernel(boxes, scores):
    masked = jnp.where(scores > _SCORE_THRESH, scores, _NEG)
    top_scores, top_ids = jax.lax.top_k(masked, _NUM_CAND)
    top_boxes = jnp.take(boxes, top_ids, axis=0)

    bc = jnp.zeros((_PAD, 4), jnp.float32).at[:_NUM_CAND].set(top_boxes)
    bt = jnp.zeros((8, _PAD), jnp.float32).at[:4].set(bc.T[:4])
    sr = jnp.full((1, _PAD), _NEG, jnp.float32).at[0, :_NUM_CAND].set(top_scores)

    out = pl.pallas_call(
        _nms_kernel,
        out_shape=jax.ShapeDtypeStruct((128, 8), jnp.float32),
        scratch_shapes=[pltpu.VMEM((_PAD, _PAD), jnp.float32)],
    )(bc, bt, sr)
    return out[:_MAX_DET, :5]


# speedup vs baseline: 115.4995x; 4.0702x over previous
import jax
import jax.numpy as jnp
from jax.experimental import pallas as pl
from jax.experimental.pallas import tpu as pltpu

_SCORE_THRESH = 0.05
_NUM_CAND = 1000
_PAD = 1024
_TILE = 256
_NMS_THRESH = 0.5
_MAX_DET = 100
_NEG = -1e9


def _nms_kernel(bc_ref, bt_ref, sr_ref, out_ref):
    f32 = jnp.float32
    x1r = bt_ref[0:1, :]
    y1r = bt_ref[1:2, :]
    x2r = bt_ref[2:3, :]
    y2r = bt_ref[3:4, :]
    area_r = (x2r - x1r) * (y2r - y1r)

    scores = sr_ref[...]
    idx = jax.lax.broadcasted_iota(jnp.int32, (1, _PAD), 1)
    keep = (scores > -1e8).astype(f32)

    dn_mm = (((1,), (0,)), ((), ()))

    for t in range(_PAD // _TILE):
        r0 = t * _TILE
        x1c = bc_ref[pl.ds(r0, _TILE), 0:1]
        y1c = bc_ref[pl.ds(r0, _TILE), 1:2]
        x2c = bc_ref[pl.ds(r0, _TILE), 2:3]
        y2c = bc_ref[pl.ds(r0, _TILE), 3:4]
        area_c = (x2c - x1c) * (y2c - y1c)
        w = jnp.maximum(jnp.minimum(x2c, x2r) - jnp.maximum(x1c, x1r), 0.0)
        h = jnp.maximum(jnp.minimum(y2c, y2r) - jnp.maximum(y1c, y1r), 0.0)
        inter = w * h
        union = area_c + area_r - inter
        over = (inter / jnp.maximum(union, 1e-9) > _NMS_THRESH).astype(f32)

        sub = jax.lax.slice(over, (0, r0), (_TILE, r0 + _TILE))
        li = jax.lax.broadcasted_iota(jnp.int32, (_TILE, _TILE), 0)
        lj = jax.lax.broadcasted_iota(jnp.int32, (_TILE, _TILE), 1)
        m = jnp.where(li < lj, sub, 0.0)
        valid_t = jax.lax.slice(keep, (0, r0), (1, r0 + _TILE))

        def cond(c):
            return c[1]

        def body(c):
            kt = c[0]
            sup = jax.lax.dot_general(kt, m, dn_mm, preferred_element_type=f32)
            ktn = jnp.where(sup > 0.0, 0.0, valid_t)
            return (ktn, jnp.any(ktn != kt))

        kt, _ = jax.lax.while_loop(cond, body, (valid_t, True))

        parts = []
        if r0 > 0:
            parts.append(jax.lax.slice(keep, (0, 0), (1, r0)))
        parts.append(kt)
        if r0 + _TILE < _PAD:
            parts.append(jax.lax.slice(keep, (0, r0 + _TILE), (1, _PAD)))
        keep = jnp.concatenate(parts, axis=1)
        sup_all = jax.lax.dot_general(kt, over, dn_mm, preferred_element_type=f32)
        keep = jnp.where((sup_all > 0.0) & (idx >= r0 + _TILE), 0.0, keep)

    r_i = jax.lax.broadcasted_iota(jnp.int32, (_PAD, _PAD), 0)
    c_i = jax.lax.broadcasted_iota(jnp.int32, (_PAD, _PAD), 1)
    tri = (r_i <= c_i).astype(f32)
    ck = jax.lax.dot_general(keep, tri, dn_mm, preferred_element_type=f32)
    cn = jax.lax.dot_general(1.0 - keep, tri, dn_mm, preferred_element_type=f32)
    num_keep = jnp.max(ck)
    pos = jnp.where(keep > 0.0, ck - 1.0, num_keep + cn - 1.0)
    posi = pos.astype(jnp.int32)

    o_i = jax.lax.broadcasted_iota(jnp.int32, (128, _PAD), 0)
    onehot = (posi == o_i).astype(f32)
    dnt = (((1,), (1,)), ((), ()))
    out_boxes = jax.lax.dot_general(onehot, bt_ref[...], dnt,
                                    preferred_element_type=f32)
    fs = jnp.where(keep > 0.0, scores, _NEG)
    out_scores = jax.lax.dot_general(onehot, fs, dnt,
                                     preferred_element_type=f32)
    col = jax.lax.broadcasted_iota(jnp.int32, (128, 8), 1)
    out_ref[...] = jnp.where(col < 4, out_boxes,
                             jnp.where(col == 4, out_scores, 0.0))


def kernel(boxes, scores):
    masked = jnp.where(scores > _SCORE_THRESH, scores, _NEG)
    top_scores, top_ids = jax.lax.top_k(masked, _NUM_CAND)
    top_boxes = jnp.take(boxes, top_ids, axis=0)

    bc = jnp.zeros((_PAD, 4), jnp.float32).at[:_NUM_CAND].set(top_boxes)
    bt = jnp.zeros((8, _PAD), jnp.float32).at[:4].set(bc.T[:4])
    sr = jnp.full((1, _PAD), _NEG, jnp.float32).at[0, :_NUM_CAND].set(top_scores)

    out = pl.pallas_call(
        _nms_kernel,
        out_shape=jax.ShapeDtypeStruct((128, 8), jnp.float32),
    )(bc, bt, sr)
    return out[:_MAX_DET, :5]


# speedup vs baseline: 118.6057x; 1.0269x over previous
import jax
import jax.numpy as jnp
from jax.experimental import pallas as pl
from jax.experimental.pallas import tpu as pltpu

_SCORE_THRESH = 0.05
_NUM_CAND = 1000
_PAD = 1024
_TILE = 512
_NMS_THRESH = 0.5
_MAX_DET = 100
_NEG = -1e9


def _nms_kernel(bc_ref, bt_ref, sr_ref, out_ref):
    f32 = jnp.float32
    x1r = bt_ref[0:1, :]
    y1r = bt_ref[1:2, :]
    x2r = bt_ref[2:3, :]
    y2r = bt_ref[3:4, :]
    area_r = (x2r - x1r) * (y2r - y1r)

    scores = sr_ref[...]
    idx = jax.lax.broadcasted_iota(jnp.int32, (1, _PAD), 1)
    keep = (scores > -1e8).astype(f32)

    dn_mm = (((1,), (0,)), ((), ()))

    for t in range(_PAD // _TILE):
        r0 = t * _TILE
        x1c = bc_ref[pl.ds(r0, _TILE), 0:1]
        y1c = bc_ref[pl.ds(r0, _TILE), 1:2]
        x2c = bc_ref[pl.ds(r0, _TILE), 2:3]
        y2c = bc_ref[pl.ds(r0, _TILE), 3:4]
        area_c = (x2c - x1c) * (y2c - y1c)
        w = jnp.maximum(jnp.minimum(x2c, x2r) - jnp.maximum(x1c, x1r), 0.0)
        h = jnp.maximum(jnp.minimum(y2c, y2r) - jnp.maximum(y1c, y1r), 0.0)
        inter = w * h
        union = area_c + area_r - inter
        over = (inter / jnp.maximum(union, 1e-9) > _NMS_THRESH).astype(f32)

        sub = jax.lax.slice(over, (0, r0), (_TILE, r0 + _TILE))
        li = jax.lax.broadcasted_iota(jnp.int32, (_TILE, _TILE), 0)
        lj = jax.lax.broadcasted_iota(jnp.int32, (_TILE, _TILE), 1)
        m = jnp.where(li < lj, sub, 0.0)
        valid_t = jax.lax.slice(keep, (0, r0), (1, r0 + _TILE))

        def cond(c):
            return c[1]

        def body(c):
            kt = c[0]
            sup = jax.lax.dot_general(kt, m, dn_mm, preferred_element_type=f32)
            ktn = jnp.where(sup > 0.0, 0.0, valid_t)
            return (ktn, jnp.any(ktn != kt))

        kt, _ = jax.lax.while_loop(cond, body, (valid_t, True))

        parts = []
        if r0 > 0:
            parts.append(jax.lax.slice(keep, (0, 0), (1, r0)))
        parts.append(kt)
        if r0 + _TILE < _PAD:
            parts.append(jax.lax.slice(keep, (0, r0 + _TILE), (1, _PAD)))
        keep = jnp.concatenate(parts, axis=1)
        sup_all = jax.lax.dot_general(kt, over, dn_mm, preferred_element_type=f32)
        keep = jnp.where((sup_all > 0.0) & (idx >= r0 + _TILE), 0.0, keep)

    r_i = jax.lax.broadcasted_iota(jnp.int32, (_PAD, _PAD), 0)
    c_i = jax.lax.broadcasted_iota(jnp.int32, (_PAD, _PAD), 1)
    tri = (r_i <= c_i).astype(f32)
    ck = jax.lax.dot_general(keep, tri, dn_mm, preferred_element_type=f32)
    cn = jax.lax.dot_general(1.0 - keep, tri, dn_mm, preferred_element_type=f32)
    num_keep = jnp.max(ck)
    pos = jnp.where(keep > 0.0, ck - 1.0, num_keep + cn - 1.0)
    posi = pos.astype(jnp.int32)

    o_i = jax.lax.broadcasted_iota(jnp.int32, (128, _PAD), 0)
    onehot = (posi == o_i).astype(f32)
    dnt = (((1,), (1,)), ((), ()))
    out_boxes = jax.lax.dot_general(onehot, bt_ref[...], dnt,
                                    preferred_element_type=f32)
    fs = jnp.where(keep > 0.0, scores, _NEG)
    out_scores = jax.lax.dot_general(onehot, fs, dnt,
                                     preferred_element_type=f32)
    col = jax.lax.broadcasted_iota(jnp.int32, (128, 8), 1)
    out_ref[...] = jnp.where(col < 4, out_boxes,
                             jnp.where(col == 4, out_scores, 0.0))


def kernel(boxes, scores):
    masked = jnp.where(scores > _SCORE_THRESH, scores, _NEG)
    top_scores, top_ids = jax.lax.top_k(masked, _NUM_CAND)
    top_boxes = jnp.take(boxes, top_ids, axis=0)

    bc = jnp.zeros((_PAD, 4), jnp.float32).at[:_NUM_CAND].set(top_boxes)
    bt = jnp.zeros((8, _PAD), jnp.float32).at[:4].set(bc.T[:4])
    sr = jnp.full((1, _PAD), _NEG, jnp.float32).at[0, :_NUM_CAND].set(top_scores)

    out = pl.pallas_call(
        _nms_kernel,
        out_shape=jax.ShapeDtypeStruct((128, 8), jnp.float32),
    )(bc, bt, sr)
    return out[:_MAX_DET, :5]
